# Initial kernel scaffold; baseline (speedup 1.0000x reference)
import jax
import jax.numpy as jnp
from jax.experimental import pallas as pl


def kernel(data, treatment_idx, outcome_idx, adjustment_set):
    raise NotImplementedError("write your pallas kernel here")


# speedup vs baseline: 9.0752x; 9.0752x over previous
import functools

import jax
import jax.numpy as jnp
from jax import lax
from jax.experimental import pallas as pl
from jax.experimental.pallas import tpu as pltpu
from jax.experimental.pallas import tpu_sc as plsc

N_ROWS = 320000
N_COLS = 128
NC = 2
NS = 16
L = 16
NW = NC * NS
ROWS_PER_TILE = N_ROWS // NW
CHUNK = 400
NCHUNKS = ROWS_PER_TILE // CHUNK
GROUPS = CHUNK // L


def _sc_body(data_hbm, cols_hbm, out_hbm, chunk_v, acc_v, cols_v):
    wid = lax.axis_index("s") * NC + lax.axis_index("c")
    base = wid * ROWS_PER_TILE
    pltpu.sync_copy(cols_hbm, cols_v)

    zero = jnp.zeros((L,), jnp.float32)
    for s in range(16):
        acc_v[s, :] = zero

    colsvec = cols_v[...]
    cvec = [jnp.broadcast_to(colsvec[j], (L,)) for j in range(4)]
    iota = lax.iota(jnp.int32, L)
    one = jnp.ones((L,), jnp.float32)

    def chunk_body(ci, carry):
        pltpu.sync_copy(data_hbm.at[pl.ds(base + ci * CHUNK, CHUNK)], chunk_v)

        def group_body(g, c2):
            rows = g * L + iota
            a0 = plsc.load_gather(chunk_v, [rows, cvec[0]])
            a1 = plsc.load_gather(chunk_v, [rows, cvec[1]])
            a2 = plsc.load_gather(chunk_v, [rows, cvec[2]])
            y = plsc.load_gather(chunk_v, [rows, cvec[3]])
            sid = a0 + 2.0 * a1 + 4.0 * a2
            for s in range(8):
                m = sid == float(s)
                plsc.addupdate(acc_v.at[s, :], jnp.where(m, y, zero))
                plsc.addupdate(acc_v.at[s + 8, :], jnp.where(m, one, zero))
            return c2

        lax.fori_loop(0, GROUPS, group_body, 0)
        return carry

    lax.fori_loop(0, NCHUNKS, chunk_body, 0)
    pltpu.sync_copy(acc_v, out_hbm.at[wid])


def _make_sc_call(interpret=False):
    return pl.kernel(
        _sc_body,
        out_type=jax.ShapeDtypeStruct((NW, 16, L), jnp.float32),
        mesh=plsc.VectorSubcoreMesh(
            core_axis_name="c", subcore_axis_name="s",
            num_cores=NC, num_subcores=NS),
        scratch_types=[
            pltpu.VMEM((CHUNK, N_COLS), jnp.float32),
            pltpu.VMEM((16, L), jnp.float32),
            pltpu.VMEM((L,), jnp.int32),
        ],
        compiler_params=pltpu.CompilerParams(needs_layout_passes=False),
        interpret=interpret,
    )


def _combine_body(p_ref, o_ref):
    t = jnp.sum(p_ref[...], axis=1, keepdims=True)
    sums = t[0:8]
    counts = t[8:16]
    means = sums / jnp.maximum(counts, 1.0)
    effects = jnp.where(counts > 0, means * counts / float(N_ROWS), 0.0)
    o_ref[0, 0] = jnp.sum(effects)


_combine = pl.pallas_call(
    _combine_body,
    out_shape=jax.ShapeDtypeStruct((1, 1), jnp.float32),
    in_specs=[pl.BlockSpec(memory_space=pltpu.VMEM)],
    out_specs=pl.BlockSpec(memory_space=pltpu.SMEM),
)


def kernel(data, treatment_idx, outcome_idx, adjustment_set):
    cols = jnp.concatenate([
        adjustment_set.astype(jnp.int32).reshape(3),
        jnp.asarray(outcome_idx, jnp.int32).reshape(1),
        jnp.zeros((L - 4,), jnp.int32),
    ])
    partials = _make_sc_call()(data, cols)
    p = partials.transpose(1, 0, 2).reshape(16, NW * L)
    return _combine(p)[0, 0]


# speedup vs baseline: 11.2597x; 1.2407x over previous
import functools

import jax
import jax.numpy as jnp
from jax import lax
from jax.experimental import pallas as pl
from jax.experimental.pallas import tpu as pltpu
from jax.experimental.pallas import tpu_sc as plsc

N_ROWS = 320000
N_COLS = 128
NC = 2
NS = 16
L = 16
NW = NC * NS
ROWS_PER_TILE = N_ROWS // NW
CHUNK = 400
NCHUNKS = ROWS_PER_TILE // CHUNK
GROUPS = CHUNK // L


def _sc_body(data_hbm, cols_hbm, out_hbm, buf0, buf1, acc_v, cols_v,
             sem0, sem1):
    wid = lax.axis_index("s") * NC + lax.axis_index("c")
    base = wid * ROWS_PER_TILE
    pltpu.sync_copy(cols_hbm, cols_v)
    bufs = (buf0, buf1)
    sems = (sem0, sem1)

    zero = jnp.zeros((L,), jnp.float32)
    one = jnp.ones((L,), jnp.float32)

    colsvec = cols_v[...]
    cvec = [jnp.broadcast_to(colsvec[j], (L,)) for j in range(4)]
    iota = lax.iota(jnp.int32, L)

    def _chunk_slice(ci):
        return data_hbm.at[pl.ds(base + ci * CHUNK, CHUNK)]

    def _accumulate(buf, rows, acc):
        a0 = plsc.load_gather(buf, [rows, cvec[0]])
        a1 = plsc.load_gather(buf, [rows, cvec[1]])
        a2 = plsc.load_gather(buf, [rows, cvec[2]])
        y = plsc.load_gather(buf, [rows, cvec[3]])
        sid = a0 + 2.0 * a1 + 4.0 * a2
        acc = list(acc)
        for s in range(8):
            m = sid == float(s)
            acc[s] = acc[s] + jnp.where(m, y, zero)
            acc[s + 8] = acc[s + 8] + jnp.where(m, one, zero)
        return tuple(acc)

    def _process(b, acc):
        def group_body(g, a, _buf=bufs[b]):
            return _accumulate(_buf, g * L + iota, a)
        return lax.fori_loop(0, GROUPS, group_body, acc)

    pltpu.async_copy(_chunk_slice(0), buf0, sem0)

    def chunk_pair(cp, acc):
        for b in range(2):
            ci = cp * 2 + b
            nb = 1 - b
            pltpu.async_copy(_chunk_slice(ci + 1), bufs[nb], sems[nb])
            pltpu.make_async_copy(_chunk_slice(ci), bufs[b], sems[b]).wait()
            acc = _process(b, acc)
        return acc

    acc0 = tuple(zero for _ in range(16))
    acc = lax.fori_loop(0, NCHUNKS // 2, chunk_pair, acc0)
    pltpu.make_async_copy(_chunk_slice(NCHUNKS - 1), bufs[0], sems[0]).wait()
    acc = _process(0, acc)
    for s in range(16):
        acc_v[s, :] = acc[s]
    pltpu.sync_copy(acc_v, out_hbm.at[wid])


def _make_sc_call(interpret=False):
    return pl.kernel(
        _sc_body,
        out_type=jax.ShapeDtypeStruct((NW, 16, L), jnp.float32),
        mesh=plsc.VectorSubcoreMesh(
            core_axis_name="c", subcore_axis_name="s",
            num_cores=NC, num_subcores=NS),
        scratch_types=[
            pltpu.VMEM((CHUNK, N_COLS), jnp.float32),
            pltpu.VMEM((CHUNK, N_COLS), jnp.float32),
            pltpu.VMEM((16, L), jnp.float32),
            pltpu.VMEM((L,), jnp.int32),
            pltpu.SemaphoreType.DMA,
            pltpu.SemaphoreType.DMA,
        ],
        compiler_params=pltpu.CompilerParams(needs_layout_passes=False),
        interpret=interpret,
    )


def _combine_body(p_ref, o_ref):
    t = jnp.sum(p_ref[...], axis=1, keepdims=True)
    sums = t[0:8]
    counts = t[8:16]
    means = sums / jnp.maximum(counts, 1.0)
    effects = jnp.where(counts > 0, means * counts / float(N_ROWS), 0.0)
    o_ref[0, 0] = jnp.sum(effects)


_combine = pl.pallas_call(
    _combine_body,
    out_shape=jax.ShapeDtypeStruct((1, 1), jnp.float32),
    in_specs=[pl.BlockSpec(memory_space=pltpu.VMEM)],
    out_specs=pl.BlockSpec(memory_space=pltpu.SMEM),
)


def kernel(data, treatment_idx, outcome_idx, adjustment_set):
    cols = jnp.concatenate([
        adjustment_set.astype(jnp.int32).reshape(3),
        jnp.asarray(outcome_idx, jnp.int32).reshape(1),
        jnp.zeros((L - 4,), jnp.int32),
    ])
    partials = _make_sc_call()(data, cols)
    p = partials.transpose(1, 0, 2).reshape(16, NW * L)
    return _combine(p)[0, 0]


# speedup vs baseline: 13.4595x; 1.1954x over previous
import functools

import jax
import jax.numpy as jnp
from jax import lax
from jax.experimental import pallas as pl
from jax.experimental.pallas import tpu as pltpu
from jax.experimental.pallas import tpu_sc as plsc

N_ROWS = 320000
N_COLS = 128
NC = 2
NS = 16
L = 16
NW = NC * NS
VR = 16
VPR = N_COLS // VR
ROWS_PER_TILE = N_ROWS // NW
RPB = 128
NBLK = ROWS_PER_TILE // RPB
TAIL = ROWS_PER_TILE - NBLK * RPB
BPC = 3
CHUNK = RPB * BPC
NCHUNKS = NBLK // BPC
GROUPS = CHUNK // L


def _sc_body(data_hbm, cols_hbm, out_hbm, buf0, buf1, tail_v, idx_v, acc_v,
             cols_v, sem0, sem1, sem_t):
    wid = lax.axis_index("s") * NC + lax.axis_index("c")
    base = wid * ROWS_PER_TILE
    pltpu.sync_copy(cols_hbm, cols_v)
    bufs = (buf0, buf1)
    sems = (sem0, sem1)

    zero = jnp.zeros((L,), jnp.float32)
    one = jnp.ones((L,), jnp.float32)
    iota = lax.iota(jnp.int32, L)

    colsvec = cols_v[...]
    cgran = [lax.shift_right_logical(colsvec[j], 4) for j in range(4)]
    clane = [jnp.broadcast_to(jnp.bitwise_and(colsvec[j], 15), (L,))
             for j in range(4)]

    for j in range(4):
        vtail = (base + NBLK * RPB + iota) * VPR + cgran[j]
        pltpu.async_copy(data_hbm.at[vtail], tail_v.at[j], sem_t)

    jfull = [jnp.full((L,), j, jnp.int32) for j in range(4)]

    def build_block(b, _):
        for g8 in range(RPB // L):
            rows = b * RPB + g8 * L + iota
            vrow = (base + rows) * VPR
            for j in range(4):
                plsc.store_scatter(idx_v, [jfull[j], rows], vrow + cgran[j])
        return 0

    lax.fori_loop(0, NBLK, build_block, 0)

    def _dmas(ci, bi):
        out = []
        for j in range(4):
            for b in range(BPC):
                blk0 = (ci * BPC + b) * RPB
                src = data_hbm.at[idx_v.at[j, pl.ds(blk0, RPB)]]
                dst = bufs[bi].at[j, pl.ds(b * RPB, RPB)]
                out.append((src, dst))
        return out

    def _start_chunk(ci, bi):
        for src, dst in _dmas(ci, bi):
            pltpu.async_copy(src, dst, sems[bi])

    def _wait_chunk(ci, bi):
        for src, dst in _dmas(ci, bi):
            pltpu.make_async_copy(src, dst, sems[bi]).wait()

    def _accumulate(bufy, bufa, g, acc):
        rows = g * L + iota
        a0 = plsc.load_gather(bufa[0], [rows, clane[0]])
        a1 = plsc.load_gather(bufa[1], [rows, clane[1]])
        a2 = plsc.load_gather(bufa[2], [rows, clane[2]])
        y = plsc.load_gather(bufy, [rows, clane[3]])
        sid = a0 + 2.0 * a1 + 4.0 * a2
        acc = list(acc)
        for s in range(8):
            m = sid == float(s)
            acc[s] = acc[s] + jnp.where(m, y, zero)
            acc[s + 8] = acc[s + 8] + jnp.where(m, one, zero)
        return tuple(acc)

    def _process(bi, acc):
        def group_body(g, a, _buf=bufs[bi]):
            return _accumulate(_buf.at[3], [_buf.at[0], _buf.at[1],
                                            _buf.at[2]], g, a)
        return lax.fori_loop(0, GROUPS, group_body, acc)

    _start_chunk(0, 0)

    def chunk_pair(cp, acc):
        for b in range(2):
            ci = cp * 2 + b
            nb = 1 - b

            @pl.when(ci + 1 < NCHUNKS)
            def _():
                _start_chunk(ci + 1, nb)

            _wait_chunk(ci, b)
            acc = _process(b, acc)
        return acc

    acc0 = tuple(zero for _ in range(16))
    acc = lax.fori_loop(0, NCHUNKS // 2, chunk_pair, acc0)

    for j in range(4):
        vtail = (base + NBLK * RPB + iota) * VPR + cgran[j]
        pltpu.make_async_copy(data_hbm.at[vtail], tail_v.at[j], sem_t).wait()
    acc = _accumulate(tail_v.at[3], [tail_v.at[0], tail_v.at[1],
                                     tail_v.at[2]], 0, acc)

    for s in range(16):
        acc_v[s, :] = acc[s]
    pltpu.sync_copy(acc_v, out_hbm.at[wid])


def _make_sc_call(interpret=False):
    return pl.kernel(
        _sc_body,
        out_type=jax.ShapeDtypeStruct((NW, 16, L), jnp.float32),
        mesh=plsc.VectorSubcoreMesh(
            core_axis_name="c", subcore_axis_name="s",
            num_cores=NC, num_subcores=NS),
        scratch_types=[
            pltpu.VMEM((4, CHUNK, VR), jnp.float32),
            pltpu.VMEM((4, CHUNK, VR), jnp.float32),
            pltpu.VMEM((4, TAIL, VR), jnp.float32),
            pltpu.VMEM((4, NBLK * RPB), jnp.int32),
            pltpu.VMEM((16, L), jnp.float32),
            pltpu.VMEM((L,), jnp.int32),
            pltpu.SemaphoreType.DMA,
            pltpu.SemaphoreType.DMA,
            pltpu.SemaphoreType.DMA,
        ],
        compiler_params=pltpu.CompilerParams(
            needs_layout_passes=False, use_tc_tiling_on_sc=False),
        interpret=interpret,
    )


def _combine_body(p_ref, o_ref):
    t = jnp.sum(p_ref[...], axis=1, keepdims=True)
    sums = t[0:8]
    counts = t[8:16]
    means = sums / jnp.maximum(counts, 1.0)
    effects = jnp.where(counts > 0, means * counts / float(N_ROWS), 0.0)
    o_ref[0, 0] = jnp.sum(effects)


_combine = pl.pallas_call(
    _combine_body,
    out_shape=jax.ShapeDtypeStruct((1, 1), jnp.float32),
    in_specs=[pl.BlockSpec(memory_space=pltpu.VMEM)],
    out_specs=pl.BlockSpec(memory_space=pltpu.SMEM),
)


def kernel(data, treatment_idx, outcome_idx, adjustment_set):
    cols = jnp.concatenate([
        adjustment_set.astype(jnp.int32).reshape(3),
        jnp.asarray(outcome_idx, jnp.int32).reshape(1),
        jnp.zeros((L - 4,), jnp.int32),
    ])
    data16 = data.reshape(N_ROWS * VPR, VR)
    partials = _make_sc_call()(data16, cols)
    p = partials.transpose(1, 0, 2).reshape(16, NW * L)
    return _combine(p)[0, 0]


# speedup vs baseline: 13.9720x; 1.0381x over previous
import functools

import jax
import jax.numpy as jnp
from jax import lax
from jax.experimental import pallas as pl
from jax.experimental.pallas import tpu as pltpu
from jax.experimental.pallas import tpu_sc as plsc

N_ROWS = 320000
N_COLS = 128
NC = 2
NS = 16
L = 16
NW = NC * NS
VR = 16
VPR = N_COLS // VR
ROWS_PER_TILE = N_ROWS // NW
RPB = 128
NBLK = ROWS_PER_TILE // RPB
TAIL = ROWS_PER_TILE - NBLK * RPB
BPC = 3
CHUNK = RPB * BPC
NCHUNKS = NBLK // BPC
GROUPS = CHUNK // L


def _sc_body(data_hbm, cols_hbm, out_hbm, buf0, buf1, tail_v, idx_v, acc_v,
             cols_v, sem0, sem1, sem_t):
    wid = lax.axis_index("s") * NC + lax.axis_index("c")
    base = wid * ROWS_PER_TILE
    pltpu.sync_copy(cols_hbm, cols_v)
    bufs = (buf0, buf1)
    sems = (sem0, sem1)

    zero = jnp.zeros((L,), jnp.float32)
    one = jnp.ones((L,), jnp.float32)
    iota = lax.iota(jnp.int32, L)

    colsvec = cols_v[...]
    cgran = [lax.shift_right_logical(colsvec[j], 4) for j in range(4)]
    clane = [jnp.broadcast_to(jnp.bitwise_and(colsvec[j], 15), (L,))
             for j in range(4)]

    for j in range(4):
        vtail = (base + NBLK * RPB + iota) * VPR + cgran[j]
        pltpu.async_copy(data_hbm.at[vtail], tail_v.at[j], sem_t)

    jfull = [jnp.full((L,), j, jnp.int32) for j in range(4)]

    def build_block(b, _):
        for g8 in range(RPB // L):
            rows = b * RPB + g8 * L + iota
            vrow = (base + rows) * VPR
            for j in range(4):
                plsc.store_scatter(idx_v, [jfull[j], rows], vrow + cgran[j])
        return 0

    lax.fori_loop(0, BPC, build_block, 0)

    def _dmas(ci, bi):
        out = []
        for j in range(4):
            for b in range(BPC):
                blk0 = (ci * BPC + b) * RPB
                src = data_hbm.at[idx_v.at[j, pl.ds(blk0, RPB)]]
                dst = bufs[bi].at[j, pl.ds(b * RPB, RPB)]
                out.append((src, dst))
        return out

    def _start_chunk(ci, bi):
        for src, dst in _dmas(ci, bi):
            pltpu.async_copy(src, dst, sems[bi])

    def _wait_chunk(ci, bi):
        for src, dst in _dmas(ci, bi):
            pltpu.make_async_copy(src, dst, sems[bi]).wait()

    def _accumulate(bufy, bufa, g, acc):
        rows = g * L + iota
        a0 = plsc.load_gather(bufa[0], [rows, clane[0]])
        a1 = plsc.load_gather(bufa[1], [rows, clane[1]])
        a2 = plsc.load_gather(bufa[2], [rows, clane[2]])
        y = plsc.load_gather(bufy, [rows, clane[3]])
        sid = a0 + 2.0 * a1 + 4.0 * a2
        acc = list(acc)
        for s in range(8):
            m = sid == float(s)
            acc[s] = acc[s] + jnp.where(m, y, zero)
            acc[s + 8] = acc[s + 8] + jnp.where(m, one, zero)
        return tuple(acc)

    def _process(bi, acc):
        def group_body(g, a, _buf=bufs[bi]):
            return _accumulate(_buf.at[3], [_buf.at[0], _buf.at[1],
                                            _buf.at[2]], g, a)
        return lax.fori_loop(0, GROUPS, group_body, acc)

    _start_chunk(0, 0)
    lax.fori_loop(BPC, NBLK, build_block, 0)

    def chunk_pair(cp, acc):
        for b in range(2):
            ci = cp * 2 + b
            nb = 1 - b

            @pl.when(ci + 1 < NCHUNKS)
            def _():
                _start_chunk(ci + 1, nb)

            _wait_chunk(ci, b)
            acc = _process(b, acc)
        return acc

    acc0 = tuple(zero for _ in range(16))
    acc = lax.fori_loop(0, NCHUNKS // 2, chunk_pair, acc0)

    for j in range(4):
        vtail = (base + NBLK * RPB + iota) * VPR + cgran[j]
        pltpu.make_async_copy(data_hbm.at[vtail], tail_v.at[j], sem_t).wait()
    acc = _accumulate(tail_v.at[3], [tail_v.at[0], tail_v.at[1],
                                     tail_v.at[2]], 0, acc)

    for s in range(16):
        acc_v[s, :] = acc[s]
    pltpu.sync_copy(acc_v, out_hbm.at[wid])


def _make_sc_call(interpret=False):
    return pl.kernel(
        _sc_body,
        out_type=jax.ShapeDtypeStruct((NW, 16, L), jnp.float32),
        mesh=plsc.VectorSubcoreMesh(
            core_axis_name="c", subcore_axis_name="s",
            num_cores=NC, num_subcores=NS),
        scratch_types=[
            pltpu.VMEM((4, CHUNK, VR), jnp.float32),
            pltpu.VMEM((4, CHUNK, VR), jnp.float32),
            pltpu.VMEM((4, TAIL, VR), jnp.float32),
            pltpu.VMEM((4, NBLK * RPB), jnp.int32),
            pltpu.VMEM((16, L), jnp.float32),
            pltpu.VMEM((L,), jnp.int32),
            pltpu.SemaphoreType.DMA,
            pltpu.SemaphoreType.DMA,
            pltpu.SemaphoreType.DMA,
        ],
        compiler_params=pltpu.CompilerParams(
            needs_layout_passes=False, use_tc_tiling_on_sc=False),
        interpret=interpret,
    )


def _combine_body(p_ref, o_ref):
    acc = p_ref[0]
    for i in range(1, NW):
        acc = acc + p_ref[i]
    t = jnp.sum(acc, axis=1, keepdims=True)
    sums = t[0:8]
    counts = t[8:16]
    means = sums / jnp.maximum(counts, 1.0)
    effects = jnp.where(counts > 0, means * counts / float(N_ROWS), 0.0)
    o_ref[0, 0] = jnp.sum(effects)


_combine = pl.pallas_call(
    _combine_body,
    out_shape=jax.ShapeDtypeStruct((1, 1), jnp.float32),
    in_specs=[pl.BlockSpec(memory_space=pltpu.VMEM)],
    out_specs=pl.BlockSpec(memory_space=pltpu.SMEM),
)


def kernel(data, treatment_idx, outcome_idx, adjustment_set):
    cols = jnp.concatenate([
        adjustment_set.astype(jnp.int32).reshape(3),
        jnp.asarray(outcome_idx, jnp.int32).reshape(1),
        jnp.zeros((L - 4,), jnp.int32),
    ])
    data16 = data.reshape(N_ROWS * VPR, VR)
    partials = _make_sc_call()(data16, cols)
    return _combine(partials)[0, 0]
